# baseline (speedup 1.0000x reference)
import functools

import jax
import jax.numpy as jnp
from jax import lax
from jax.experimental import pallas as pl
from jax.experimental.pallas import tpu as pltpu
from jax.experimental.pallas import tpu_sc as plsc

_NUM_WORKERS = 32
_CHUNK = 128
_BUCKET_BITS = 11
_SLICE = 1 << _BUCKET_BITS
_POS_BITS = 15
_SCAN_BASE = 1


@functools.partial(jax.jit, static_argnums=(2, 3))
def _sc_gather(table, idx_flat, n, d):
    v = table.shape[0]
    b_per_w = n // _NUM_WORKERS
    n_vec = b_per_w // 16
    n_pass = -(-v // _SLICE)
    tile_part = _SLICE // 16
    mesh = plsc.VectorSubcoreMesh(core_axis_name="c", subcore_axis_name="s")

    @functools.partial(
        pl.kernel,
        mesh=mesh,
        compiler_params=pltpu.CompilerParams(needs_layout_passes=False),
        out_type=jax.ShapeDtypeStruct((n, d), jnp.float32),
        scratch_types=[
            pltpu.VMEM((b_per_w,), jnp.int32),
            pltpu.VMEM((b_per_w + _CHUNK,), jnp.int32),
            pltpu.VMEM((64,), jnp.int32),
            pltpu.VMEM((64,), jnp.int32),
            pltpu.VMEM((64,), jnp.int32),
            pltpu.VMEM((2, _CHUNK, d), jnp.float32),
            pltpu.VMEM((2, 1, _CHUNK), jnp.int32),
            pltpu.VMEM((2, 1, _CHUNK), jnp.int32),
            pltpu.VMEM_SHARED((2, _SLICE, d), jnp.float32),
            pltpu.SemaphoreType.DMA,
            pltpu.SemaphoreType.DMA,
            pltpu.SemaphoreType.DMA,
            pltpu.SemaphoreType.DMA,
            pltpu.SemaphoreType.DMA,
        ],
    )
    def body(
        table_hbm,
        idx_hbm,
        out_hbm,
        ids_v,
        arena,
        hist_v,
        base_v,
        ptr_v,
        rows_v,
        gidx_v,
        sidx_v,
        spmem,
        ssem,
        gsem0,
        gsem1,
        wsem0,
        wsem1,
    ):
        cid = lax.axis_index("c")
        sid = lax.axis_index("s")
        wid = cid * 16 + sid
        base = pl.multiple_of(wid * b_per_w, _CHUNK)
        gsem = (gsem0, gsem1)
        wsem = (wsem0, wsem1)
        lane = lax.iota(jnp.int32, 16)

        def start_slice_load(p):
            lo = p * _SLICE
            src = jnp.minimum(lo + sid * tile_part, v - tile_part)
            src = pl.multiple_of(src, 8)
            dst = pl.multiple_of(src - lo, 8)
            pltpu.async_copy(
                table_hbm.at[pl.ds(src, tile_part)],
                spmem.at[p % 2].at[pl.ds(dst, tile_part)],
                ssem,
            )

        def wait_slice_load(p):
            pltpu.make_async_copy(
                table_hbm.at[pl.ds(0, tile_part)],
                spmem.at[p % 2].at[pl.ds(0, tile_part)],
                ssem,
            ).wait()

        def bin_ids():
            zeros = jnp.zeros((16,), jnp.int32)
            for w in range(4):
                hist_v[pl.ds(16 * w, 16)] = zeros

            def hstep(i, carry):
                b = lax.shift_right_logical(
                    ids_v[pl.ds(i * 16, 16)], _BUCKET_BITS
                )
                c, last = plsc.scan_count(b)
                h = plsc.load_gather(hist_v, [b])
                plsc.store_scatter(
                    hist_v, [b], h + c + (1 - _SCAN_BASE), mask=last
                )
                return carry

            lax.fori_loop(0, n_vec, hstep, 0)

            hw = [hist_v[pl.ds(16 * w, 16)] for w in range(4)]
            one_lane = lane == 0
            run = jnp.int32(0)
            for j in range(n_pass):
                jv = jnp.broadcast_to(jnp.int32(j), (16,))
                rv = jnp.broadcast_to(run, (16,))
                plsc.store_scatter(base_v, [jv], rv, mask=one_lane)
                plsc.store_scatter(ptr_v, [jv], rv, mask=one_lane)
                run = run + hw[j // 16][j % 16]

            def pstep(i, carry):
                tid = ids_v[pl.ds(i * 16, 16)]
                b = lax.shift_right_logical(tid, _BUCKET_BITS)
                pk = lax.shift_left(tid & (_SLICE - 1), _POS_BITS) | (
                    lane + i * 16
                )
                c, last = plsc.scan_count(b)
                pb = plsc.load_gather(ptr_v, [b])
                off = pb + (c - _SCAN_BASE)
                plsc.store_scatter(arena, [off], pk)
                plsc.store_scatter(ptr_v, [b], off + 1, mask=last)
                return carry

            lax.fori_loop(0, n_vec, pstep, 0)

        def unpack_chunk(j, bbuf, bs, cnt, firstv):
            def step(u, carry):
                w = j * _CHUNK + u * 16
                pkv = arena[pl.ds(bs + w, 16)]
                keep = (w + lane) < cnt
                pkv = jnp.where(keep, pkv, firstv)
                gidx_v[bbuf, 0, pl.ds(u * 16, 16)] = lax.shift_right_logical(
                    pkv, _POS_BITS
                )
                sidx_v[bbuf, 0, pl.ds(u * 16, 16)] = (
                    pkv & ((1 << _POS_BITS) - 1)
                ) + base
                return carry

            lax.fori_loop(0, _CHUNK // 16, step, 0)

        def start_chunk_gather(p, bbuf):
            pltpu.async_copy(
                spmem.at[p % 2].at[gidx_v.at[bbuf].at[0]],
                rows_v.at[bbuf],
                gsem[bbuf],
            )

        def wait_chunk_gather(p, bbuf):
            pltpu.make_async_copy(
                spmem.at[p % 2].at[pl.ds(0, _CHUNK)],
                rows_v.at[bbuf],
                gsem[bbuf],
            ).wait()

        def start_chunk_scatter(bbuf):
            pltpu.async_copy(
                rows_v.at[bbuf], out_hbm.at[sidx_v.at[bbuf].at[0]], wsem[bbuf]
            )

        def wait_chunk_scatter(bbuf):
            pltpu.make_async_copy(
                rows_v.at[bbuf], out_hbm.at[pl.ds(0, _CHUNK)], wsem[bbuf]
            ).wait()

        def run_chunks(p):
            cnt = hist_v[pl.ds(16 * (p // 16), 16)][p % 16]
            bs = base_v[pl.ds(16 * (p // 16), 16)][p % 16]
            nchunks = lax.div(cnt + _CHUNK - 1, _CHUNK)

            @pl.when(cnt > 0)
            def _():
                firstv = jnp.broadcast_to(arena[pl.ds(bs, 16)][0], (16,))
                unpack_chunk(0, 0, bs, cnt, firstv)
                start_chunk_gather(p, 0)

                def step(i, carry):
                    for bbuf in range(2):
                        j = i * 2 + bbuf

                        @pl.when(j < nchunks)
                        def _():
                            @pl.when(j + 1 < nchunks)
                            def _():
                                unpack_chunk(j + 1, 1 - bbuf, bs, cnt, firstv)
                                start_chunk_gather(p, 1 - bbuf)

                            wait_chunk_gather(p, bbuf)
                            start_chunk_scatter(bbuf)
                            wait_chunk_scatter(bbuf)

                    return carry

                lax.fori_loop(0, lax.div(nchunks + 1, 2), step, 0)

        pltpu.sync_copy(idx_hbm.at[pl.ds(base, b_per_w)], ids_v)
        start_slice_load(0)
        bin_ids()
        wait_slice_load(0)
        plsc.subcore_barrier()

        for p in range(n_pass):
            if p + 1 < n_pass:
                start_slice_load(p + 1)
            run_chunks(p)
            if p + 1 < n_pass:
                wait_slice_load(p + 1)
                plsc.subcore_barrier()

    return body(table, idx_flat)


def kernel(token_ids, embedding_matrix):
    b, t = token_ids.shape
    v, d = embedding_matrix.shape
    n = b * t
    idx_flat = token_ids.reshape(n).astype(jnp.int32)
    out = _sc_gather(embedding_matrix, idx_flat, n, d)
    return out.reshape(b, t, d)


# speedup vs baseline: 1.5759x; 1.5759x over previous
import functools

import jax
import jax.numpy as jnp
from jax import lax
from jax.experimental import pallas as pl
from jax.experimental.pallas import tpu as pltpu
from jax.experimental.pallas import tpu_sc as plsc

_NUM_WORKERS = 32
_CHUNK = 128
_NBUF = 6
_LEAD = 3


@functools.partial(jax.jit, static_argnums=(2, 3))
def _sc_gather(table, idx_flat, n, d):
    b_per_w = n // _NUM_WORKERS
    steps = b_per_w // _CHUNK
    mesh = plsc.VectorSubcoreMesh(core_axis_name="c", subcore_axis_name="s")

    @functools.partial(
        pl.kernel,
        mesh=mesh,
        out_type=jax.ShapeDtypeStruct((n, d), jnp.float32),
        scratch_types=[
            pltpu.VMEM((b_per_w,), jnp.int32),
            pltpu.VMEM((_NBUF, _CHUNK, d), jnp.float32),
        ] + [pltpu.SemaphoreType.DMA] * (2 * _NBUF),
    )
    def body(table_hbm, idx_hbm, out_hbm, idx_v, rows_v, *sems):
        gsem = sems[:_NBUF]
        wsem = sems[_NBUF:]
        wid = lax.axis_index("c") * 16 + lax.axis_index("s")
        base = pl.multiple_of(wid * b_per_w, _CHUNK)
        pltpu.sync_copy(idx_hbm.at[pl.ds(base, b_per_w)], idx_v)

        def start_gather(g, b):
            off = pl.multiple_of(g * _CHUNK, _CHUNK)
            pltpu.async_copy(
                table_hbm.at[idx_v.at[pl.ds(off, _CHUNK)]], rows_v.at[b], gsem[b]
            )

        def wait_gather(b):
            pltpu.make_async_copy(
                table_hbm.at[pl.ds(0, _CHUNK)], rows_v.at[b], gsem[b]
            ).wait()

        def start_write(g, b):
            off = pl.multiple_of(g * _CHUNK, _CHUNK)
            pltpu.async_copy(rows_v.at[b], out_hbm.at[pl.ds(base + off, _CHUNK)], wsem[b])

        def wait_write(g, b):
            off = pl.multiple_of(g * _CHUNK, _CHUNK)
            pltpu.make_async_copy(
                rows_v.at[b], out_hbm.at[pl.ds(base + off, _CHUNK)], wsem[b]
            ).wait()

        for b in range(_LEAD):
            start_gather(b, b)

        def visit(g, b):
            bn = (b + _LEAD) % _NBUF
            wait_gather(b)
            start_write(g, b)

            @pl.when(g >= _LEAD)
            def _():
                wait_write(g - _LEAD, bn)

            @pl.when(g + _LEAD < steps)
            def _():
                start_gather(g + _LEAD, bn)

        def outer(i, carry):
            for b in range(_NBUF):
                visit(i * _NBUF + b, b)
            return carry

        full = steps // _NBUF
        lax.fori_loop(0, full, outer, 0)
        for g in range(full * _NBUF, steps):
            visit(g, g % _NBUF)
        for g in range(steps - _LEAD, steps):
            wait_write(g, g % _NBUF)

    return body(table, idx_flat)


def kernel(token_ids, embedding_matrix):
    b, t = token_ids.shape
    v, d = embedding_matrix.shape
    n = b * t
    idx_flat = token_ids.reshape(n).astype(jnp.int32)
    out = _sc_gather(embedding_matrix, idx_flat, n, d)
    return out.reshape(b, t, d)
